# baseline (speedup 1.0000x reference)
import functools

import jax
import jax.numpy as jnp
from jax import lax
from jax.experimental import pallas as pl
from jax.experimental.pallas import tpu as pltpu
from jax.experimental.pallas import tpu_sc as plsc

_N, _E, _D, _G, _C = 10000, 320000, 128, 64, 16
_NC, _NS = 2, 16
_NW = _NC * _NS
_EPW = _E // _NW
_K = 40
_CH = _EPW // _K
_NBUF = 5
_NDC = _N // _K

_sc_mesh = plsc.VectorSubcoreMesh(core_axis_name="c", subcore_axis_name="s")


@functools.partial(
    pl.kernel,
    out_type=jax.ShapeDtypeStruct((_NC, _N, _D), jnp.float32),
    mesh=_sc_mesh,
    scratch_types=[
        pltpu.VMEM((_EPW,), jnp.int32),
        [pltpu.VMEM((_K,), jnp.int32) for _ in range(_NBUF)],
        [pltpu.VMEM((_K, _D), jnp.float32) for _ in range(_NBUF)],
        pltpu.VMEM_SHARED((_N, _D), jnp.float32),
        [pltpu.SemaphoreType.DMA for _ in range(_NBUF)],
        [pltpu.SemaphoreType.DMA for _ in range(_NBUF)],
        [pltpu.SemaphoreType.DMA for _ in range(_NBUF)],
        pltpu.SemaphoreType.DMA,
    ],
)
def _sc_agg(x_hbm, z_hbm, src_hbm, dst_hbm, out_hbm, src_v, dstb, rows, acc,
            gsems, dsems, ssems, zsem):
    cid = lax.axis_index("c")
    sid = lax.axis_index("s")
    wid = sid * _NC + cid
    ebase = wid * _EPW

    pltpu.sync_copy(src_hbm.at[pl.ds(ebase, _EPW)], src_v)
    for b in range(_NBUF):
        pltpu.async_copy(x_hbm.at[src_v.at[pl.ds(b * _K, _K)]], rows[b],
                         gsems[b])
        pltpu.async_copy(dst_hbm.at[pl.ds(ebase + b * _K, _K)], dstb[b],
                         dsems[b])

    @pl.loop(0, (_NDC + _NS - 1) // _NS)
    def _(j):
        ch = sid + j * _NS

        @pl.when(ch < _NDC)
        def _():
            sl = pl.ds(ch * _K, _K)

            @pl.when(cid == 0)
            def _():
                pltpu.async_copy(x_hbm.at[sl], acc.at[sl], zsem)

            @pl.when(cid != 0)
            def _():
                pltpu.async_copy(z_hbm.at[sl], acc.at[sl], zsem)

    @pl.loop(0, (_NDC + _NS - 1) // _NS)
    def _(j):
        ch = sid + j * _NS

        @pl.when(ch < _NDC)
        def _():
            sl = pl.ds(ch * _K, _K)
            pltpu.make_async_copy(z_hbm.at[sl], acc.at[sl], zsem).wait()

    plsc.subcore_barrier()

    _run_main(x_hbm, src_hbm, dst_hbm, src_v, dstb, rows, acc,
              gsems, dsems, ssems, ebase)

    plsc.subcore_barrier()

    @pl.loop(0, (_NDC + _NS - 1) // _NS)
    def _(j):
        ch = sid + j * _NS

        @pl.when(ch < _NDC)
        def _():
            sl = pl.ds(ch * _K, _K)
            pltpu.async_copy(acc.at[sl], out_hbm.at[cid, sl], zsem)

    @pl.loop(0, (_NDC + _NS - 1) // _NS)
    def _(j):
        ch = sid + j * _NS

        @pl.when(ch < _NDC)
        def _():
            sl = pl.ds(ch * _K, _K)
            pltpu.make_async_copy(acc.at[sl], out_hbm.at[cid, sl],
                                  zsem).wait()


def _run_main(x_hbm, src_hbm, dst_hbm, src_v, dstb, rows, acc, gsems,
              dsems, ssems, ebase):
    @pl.loop(0, (_CH // _NBUF) * _NBUF - _NBUF, step=_NBUF)
    def _(ci):
        for b in range(_NBUF):
            cur = ci + b
            pltpu.make_async_copy(x_hbm.at[src_v.at[pl.ds(cur * _K, _K)]],
                                  rows[b], gsems[b]).wait()
            pltpu.make_async_copy(dst_hbm.at[pl.ds(ebase + cur * _K, _K)],
                                  dstb[b], dsems[b]).wait()
            pltpu.sync_copy(rows[b], acc.at[dstb[b]], add=True)
            nxt = cur + _NBUF

            @pl.when(nxt < _CH)
            def _():
                pltpu.async_copy(x_hbm.at[src_v.at[pl.ds(nxt * _K, _K)]],
                                 rows[b], gsems[b])
                pltpu.async_copy(dst_hbm.at[pl.ds(ebase + nxt * _K, _K)],
                                 dstb[b], dsems[b])

    for cur in range((_CH // _NBUF) * _NBUF - _NBUF, _CH):
        b = cur % _NBUF
        pltpu.make_async_copy(x_hbm.at[src_v.at[pl.ds(cur * _K, _K)]],
                              rows[b], gsems[b]).wait()
        pltpu.make_async_copy(dst_hbm.at[pl.ds(ebase + cur * _K, _K)],
                              dstb[b], dsems[b]).wait()
        pltpu.sync_copy(rows[b], acc.at[dstb[b]], add=True)
        nxt = cur + _NBUF
        if nxt < _CH:
            pltpu.async_copy(x_hbm.at[src_v.at[pl.ds(nxt * _K, _K)]],
                             rows[b], gsems[b])
            pltpu.async_copy(dst_hbm.at[pl.ds(ebase + nxt * _K, _K)],
                             dstb[b], dsems[b])


_BN = 10000
_NB = _N // _BN


def _mlp_body(final_relu, a0_ref, a1_ref, wa_ref, ba_ref, wb_ref,
              bb_ref, o_ref):
    h = a0_ref[...] + a1_ref[...]
    h = jnp.dot(h, wa_ref[...], preferred_element_type=jnp.float32)
    h = jnp.maximum(h + ba_ref[...], 0.0)
    h = jnp.dot(h, wb_ref[...], preferred_element_type=jnp.float32)
    h = h + bb_ref[...]
    if final_relu:
        h = jnp.maximum(h, 0.0)
    o_ref[...] = h


def _tc_mlp(a0, a1, wa, ba, wb, bb, final_relu):
    row = pl.BlockSpec((_BN, _D), lambda i: (i, 0))
    full = pl.BlockSpec((_D, _D), lambda i: (0, 0))
    vec = pl.BlockSpec((1, _D), lambda i: (0, 0))
    return pl.pallas_call(
        functools.partial(_mlp_body, final_relu),
        grid=(_NB,),
        in_specs=[row, row, full, vec, full, vec],
        out_specs=row,
        out_shape=jax.ShapeDtypeStruct((_N, _D), jnp.float32),
    )(a0, a1, wa, ba.reshape(1, _D), wb, bb.reshape(1, _D))


def _pool_body(a0_ref, a1_ref, wa_ref, ba_ref, wb_ref, bb_ref,
               batch_ref, wl_ref, bl_ref, o_ref, sums, counts):
    i = pl.program_id(0)

    @pl.when(i == 0)
    def _():
        sums[...] = jnp.zeros((_G, _D), jnp.float32)
        counts[...] = jnp.zeros((_G, _D), jnp.float32)

    h = a0_ref[...] + a1_ref[...]
    h = jnp.dot(h, wa_ref[...], preferred_element_type=jnp.float32)
    h = jnp.maximum(h + ba_ref[...], 0.0)
    h = jnp.dot(h, wb_ref[...], preferred_element_type=jnp.float32)
    h = h + bb_ref[...]

    seg = batch_ref[0]
    onehot = (lax.broadcasted_iota(jnp.int32, (_G, _BN), 0) == seg)
    onehot = onehot.astype(jnp.float32)
    sums[...] += jnp.dot(onehot, h, preferred_element_type=jnp.float32)
    counts[...] += jnp.broadcast_to(
        jnp.sum(onehot, axis=1, keepdims=True), (_G, _D))

    @pl.when(i == _NB - 1)
    def _():
        pooled = sums[...] / jnp.maximum(counts[...], 1.0)
        o_ref[...] = (jnp.dot(pooled, wl_ref[...],
                              preferred_element_type=jnp.float32)
                      + bl_ref[...])


def _tc_mlp_pool(a0, a1, wa, ba, wb, bb, batch, wl, bl):
    row = pl.BlockSpec((_BN, _D), lambda i: (i, 0))
    full = pl.BlockSpec((_D, _D), lambda i: (0, 0))
    vec = pl.BlockSpec((1, _D), lambda i: (0, 0))
    return pl.pallas_call(
        _pool_body,
        grid=(_NB,),
        in_specs=[
            row, row, full, vec, full, vec,
            pl.BlockSpec((1, 1, _BN), lambda i: (i, 0, 0)),
            pl.BlockSpec((_D, _C), lambda i: (0, 0)),
            pl.BlockSpec((1, _C), lambda i: (0, 0)),
        ],
        out_specs=pl.BlockSpec((_G, _C), lambda i: (0, 0)),
        out_shape=jax.ShapeDtypeStruct((_G, _C), jnp.float32),
        scratch_shapes=[
            pltpu.VMEM((_G, _D), jnp.float32),
            pltpu.VMEM((_G, _D), jnp.float32),
        ],
    )(a0, a1, wa, ba.reshape(1, _D), wb, bb.reshape(1, _D),
      batch.reshape(_NB, 1, _BN), wl, bl.reshape(1, _C))


def kernel(x, edge_index, batch, W1a, b1a, W1b, b1b, W2a, b2a, W2b, b2b,
           Wl, bl):
    src = edge_index[0]
    dst = edge_index[1]
    z = jnp.zeros((_N, _D), jnp.float32)

    agg = _sc_agg(x, z, src, dst)
    h1 = _tc_mlp(agg[0], agg[1], W1a, b1a, W1b, b1b, final_relu=True)
    agg = _sc_agg(h1, z, src, dst)
    h2 = _tc_mlp(agg[0], agg[1], W2a, b2a, W2b, b2b, final_relu=True)
    agg = _sc_agg(h2, z, src, dst)
    return _tc_mlp_pool(agg[0], agg[1], W2a, b2a, W2b, b2b, batch, Wl, bl)


# speedup vs baseline: 1.0094x; 1.0094x over previous
import functools

import jax
import jax.numpy as jnp
from jax import lax
from jax.experimental import pallas as pl
from jax.experimental.pallas import tpu as pltpu
from jax.experimental.pallas import tpu_sc as plsc

_N, _E, _D, _G, _C = 10000, 320000, 128, 64, 16
_NC, _NS = 2, 16
_NW = _NC * _NS
_EPW = _E // _NW
_K = 40
_CH = _EPW // _K
_NBUF = 5
_NDC = _N // _K

_sc_mesh = plsc.VectorSubcoreMesh(core_axis_name="c", subcore_axis_name="s")


@functools.partial(
    pl.kernel,
    out_type=jax.ShapeDtypeStruct((_NC, _N, _D), jnp.float32),
    mesh=_sc_mesh,
    scratch_types=[
        pltpu.VMEM((_EPW,), jnp.int32),
        [pltpu.VMEM((_K,), jnp.int32) for _ in range(_NBUF)],
        [pltpu.VMEM((_K, _D), jnp.float32) for _ in range(_NBUF)],
        pltpu.VMEM((_K, _D), jnp.float32),
        pltpu.VMEM_SHARED((_N, _D), jnp.float32),
        [pltpu.SemaphoreType.DMA for _ in range(_NBUF)],
        [pltpu.SemaphoreType.DMA for _ in range(_NBUF)],
        [pltpu.SemaphoreType.DMA for _ in range(_NBUF)],
        pltpu.SemaphoreType.DMA,
    ],
)
def _sc_agg(x_hbm, src_hbm, dst_hbm, out_hbm, src_v, dstb, rows, zbuf, acc,
            gsems, dsems, ssems, zsem):
    cid = lax.axis_index("c")
    sid = lax.axis_index("s")
    wid = sid * _NC + cid
    ebase = wid * _EPW

    pltpu.sync_copy(src_hbm.at[pl.ds(ebase, _EPW)], src_v)
    for b in range(_NBUF):
        pltpu.async_copy(x_hbm.at[src_v.at[pl.ds(b * _K, _K)]], rows[b],
                         gsems[b])
        pltpu.async_copy(dst_hbm.at[pl.ds(ebase + b * _K, _K)], dstb[b],
                         dsems[b])

    @pl.when(cid != 0)
    def _():
        @pl.loop(0, _K * (_D // 16))
        def _(i):
            r = i // (_D // 16)
            c16 = (i % (_D // 16)) * 16
            zbuf[r, pl.ds(c16, 16)] = jnp.zeros((16,), jnp.float32)

    @pl.loop(0, (_NDC + _NS - 1) // _NS)
    def _(j):
        ch = sid + j * _NS

        @pl.when(ch < _NDC)
        def _():
            sl = pl.ds(ch * _K, _K)

            @pl.when(cid == 0)
            def _():
                pltpu.async_copy(x_hbm.at[sl], acc.at[sl], zsem)

            @pl.when(cid != 0)
            def _():
                pltpu.async_copy(zbuf, acc.at[sl], zsem)

    @pl.loop(0, (_NDC + _NS - 1) // _NS)
    def _(j):
        ch = sid + j * _NS

        @pl.when(ch < _NDC)
        def _():
            sl = pl.ds(ch * _K, _K)

            @pl.when(cid == 0)
            def _():
                pltpu.make_async_copy(x_hbm.at[sl], acc.at[sl], zsem).wait()

            @pl.when(cid != 0)
            def _():
                pltpu.make_async_copy(zbuf, acc.at[sl], zsem).wait()

    plsc.subcore_barrier()

    _run_main(x_hbm, src_hbm, dst_hbm, src_v, dstb, rows, acc,
              gsems, dsems, ssems, ebase)

    plsc.subcore_barrier()

    @pl.loop(0, (_NDC + _NS - 1) // _NS)
    def _(j):
        ch = sid + j * _NS

        @pl.when(ch < _NDC)
        def _():
            sl = pl.ds(ch * _K, _K)
            pltpu.async_copy(acc.at[sl], out_hbm.at[cid, sl], zsem)

    @pl.loop(0, (_NDC + _NS - 1) // _NS)
    def _(j):
        ch = sid + j * _NS

        @pl.when(ch < _NDC)
        def _():
            sl = pl.ds(ch * _K, _K)
            pltpu.make_async_copy(acc.at[sl], out_hbm.at[cid, sl],
                                  zsem).wait()


def _run_main(x_hbm, src_hbm, dst_hbm, src_v, dstb, rows, acc, gsems,
              dsems, ssems, ebase):
    @pl.loop(0, (_CH // _NBUF) * _NBUF - _NBUF, step=_NBUF)
    def _(ci):
        for b in range(_NBUF):
            cur = ci + b
            pltpu.make_async_copy(x_hbm.at[src_v.at[pl.ds(cur * _K, _K)]],
                                  rows[b], gsems[b]).wait()
            pltpu.make_async_copy(dst_hbm.at[pl.ds(ebase + cur * _K, _K)],
                                  dstb[b], dsems[b]).wait()
            pltpu.sync_copy(rows[b], acc.at[dstb[b]], add=True)
            nxt = cur + _NBUF

            @pl.when(nxt < _CH)
            def _():
                pltpu.async_copy(x_hbm.at[src_v.at[pl.ds(nxt * _K, _K)]],
                                 rows[b], gsems[b])
                pltpu.async_copy(dst_hbm.at[pl.ds(ebase + nxt * _K, _K)],
                                 dstb[b], dsems[b])

    for cur in range((_CH // _NBUF) * _NBUF - _NBUF, _CH):
        b = cur % _NBUF
        pltpu.make_async_copy(x_hbm.at[src_v.at[pl.ds(cur * _K, _K)]],
                              rows[b], gsems[b]).wait()
        pltpu.make_async_copy(dst_hbm.at[pl.ds(ebase + cur * _K, _K)],
                              dstb[b], dsems[b]).wait()
        pltpu.sync_copy(rows[b], acc.at[dstb[b]], add=True)
        nxt = cur + _NBUF
        if nxt < _CH:
            pltpu.async_copy(x_hbm.at[src_v.at[pl.ds(nxt * _K, _K)]],
                             rows[b], gsems[b])
            pltpu.async_copy(dst_hbm.at[pl.ds(ebase + nxt * _K, _K)],
                             dstb[b], dsems[b])


_BN = 5000
_NB = _N // _BN


def _mlp_body(final_relu, a0_ref, a1_ref, wa_ref, ba_ref, wb_ref,
              bb_ref, o_ref):
    h = a0_ref[...] + a1_ref[...]
    h = jnp.dot(h, wa_ref[...], preferred_element_type=jnp.float32)
    h = jnp.maximum(h + ba_ref[...], 0.0)
    h = jnp.dot(h, wb_ref[...], preferred_element_type=jnp.float32)
    h = h + bb_ref[...]
    if final_relu:
        h = jnp.maximum(h, 0.0)
    o_ref[...] = h


def _tc_mlp(a0, a1, wa, ba, wb, bb, final_relu):
    row = pl.BlockSpec((_BN, _D), lambda i: (i, 0))
    full = pl.BlockSpec((_D, _D), lambda i: (0, 0))
    vec = pl.BlockSpec((1, _D), lambda i: (0, 0))
    return pl.pallas_call(
        functools.partial(_mlp_body, final_relu),
        grid=(_NB,),
        in_specs=[row, row, full, vec, full, vec],
        out_specs=row,
        out_shape=jax.ShapeDtypeStruct((_N, _D), jnp.float32),
    )(a0, a1, wa, ba.reshape(1, _D), wb, bb.reshape(1, _D))


def _pool_body(a0_ref, a1_ref, wa_ref, ba_ref, wb_ref, bb_ref,
               batch_ref, wl_ref, bl_ref, o_ref, sums, counts):
    i = pl.program_id(0)

    @pl.when(i == 0)
    def _():
        sums[...] = jnp.zeros((_G, _D), jnp.float32)
        counts[...] = jnp.zeros((_G, _D), jnp.float32)

    h = a0_ref[...] + a1_ref[...]
    h = jnp.dot(h, wa_ref[...], preferred_element_type=jnp.float32)
    h = jnp.maximum(h + ba_ref[...], 0.0)
    h = jnp.dot(h, wb_ref[...], preferred_element_type=jnp.float32)
    h = h + bb_ref[...]

    seg = batch_ref[0]
    onehot = (lax.broadcasted_iota(jnp.int32, (_G, _BN), 0) == seg)
    onehot = onehot.astype(jnp.float32)
    sums[...] += jnp.dot(onehot, h, preferred_element_type=jnp.float32)
    counts[...] += jnp.broadcast_to(
        jnp.sum(onehot, axis=1, keepdims=True), (_G, _D))

    @pl.when(i == _NB - 1)
    def _():
        pooled = sums[...] / jnp.maximum(counts[...], 1.0)
        o_ref[...] = (jnp.dot(pooled, wl_ref[...],
                              preferred_element_type=jnp.float32)
                      + bl_ref[...])


def _tc_mlp_pool(a0, a1, wa, ba, wb, bb, batch, wl, bl):
    row = pl.BlockSpec((_BN, _D), lambda i: (i, 0))
    full = pl.BlockSpec((_D, _D), lambda i: (0, 0))
    vec = pl.BlockSpec((1, _D), lambda i: (0, 0))
    return pl.pallas_call(
        _pool_body,
        grid=(_NB,),
        in_specs=[
            row, row, full, vec, full, vec,
            pl.BlockSpec((1, 1, _BN), lambda i: (i, 0, 0)),
            pl.BlockSpec((_D, _C), lambda i: (0, 0)),
            pl.BlockSpec((1, _C), lambda i: (0, 0)),
        ],
        out_specs=pl.BlockSpec((_G, _C), lambda i: (0, 0)),
        out_shape=jax.ShapeDtypeStruct((_G, _C), jnp.float32),
        scratch_shapes=[
            pltpu.VMEM((_G, _D), jnp.float32),
            pltpu.VMEM((_G, _D), jnp.float32),
        ],
    )(a0, a1, wa, ba.reshape(1, _D), wb, bb.reshape(1, _D),
      batch.reshape(_NB, 1, _BN), wl, bl.reshape(1, _C))


def kernel(x, edge_index, batch, W1a, b1a, W1b, b1b, W2a, b2a, W2b, b2b,
           Wl, bl):
    src = edge_index[0]
    dst = edge_index[1]

    agg = _sc_agg(x, src, dst)
    h1 = _tc_mlp(agg[0], agg[1], W1a, b1a, W1b, b1b, final_relu=True)
    agg = _sc_agg(h1, src, dst)
    h2 = _tc_mlp(agg[0], agg[1], W2a, b2a, W2b, b2b, final_relu=True)
    agg = _sc_agg(h2, src, dst)
    return _tc_mlp_pool(agg[0], agg[1], W2a, b2a, W2b, b2b, batch, Wl, bl)
